# baseline (device time: 96278 ns/iter reference)
import jax
import jax.numpy as jnp
from jax import lax
from jax.experimental import pallas as pl
from jax.experimental.pallas import tpu as pltpu

M = 4096
NC = 1024
SL = 8
LN = NC // SL
R = 256
MAXC = M // R
TAIL = MAXC
NSEM = MAXC + 1


def _cast_bf16(x):
    def body(x_ref, o_ref):
        o_ref[...] = x_ref[...].astype(jnp.bfloat16)

    return pl.pallas_call(
        body,
        grid=(16,),
        in_specs=[pl.BlockSpec((M // 16, NC), lambda i: (i, 0))],
        out_specs=pl.BlockSpec((M // 16, NC), lambda i: (i, 0)),
        out_shape=jax.ShapeDtypeStruct((M, NC), jnp.bfloat16),
    )(x)


def kernel(x, dest):
    p = lax.axis_index("x")
    order = jnp.argsort(dest, stable=True)
    xs = _cast_bf16(x)[order]

    L = jnp.sum(dest == (1 - p)).astype(jnp.int32)
    S = jnp.where(p == 0, M - L, 0)
    D = jnp.where(p == 0, 0, M - L)
    K = jnp.where(p == 0, 0, L)
    scal = jnp.stack([L, S, D, K]).astype(jnp.int32)

    xs3 = xs.reshape(M, SL, LN)

    def body(s_ref, x_ref, out_ref, send_sems, recv_sems, copy_sems):
        my_x = lax.axis_index("x")
        my_y = lax.axis_index("y")
        peer = (1 - my_x, my_y)

        L = s_ref[0]
        S = s_ref[1]
        D = s_ref[2]
        K = s_ref[3]
        keep_len = M - L
        n_full = L // R
        rem = L - n_full * R
        k_full = keep_len // R
        k_rem = keep_len - k_full * R

        barrier_sem = pltpu.get_barrier_semaphore()
        pl.semaphore_signal(
            barrier_sem, inc=1, device_id=peer,
            device_id_type=pl.DeviceIdType.MESH,
        )
        pl.semaphore_wait(barrier_sem, 1)

        def swap_desc(src_off, dst_off, sem_i):
            return pltpu.make_async_remote_copy(
                src_ref=x_ref.at[pl.ds(src_off, R)],
                dst_ref=out_ref.at[pl.ds(dst_off, R)],
                send_sem=send_sems.at[sem_i],
                recv_sem=recv_sems.at[sem_i],
                device_id=peer,
                device_id_type=pl.DeviceIdType.MESH,
            )

        def keep_desc(off, sem_i):
            return pltpu.make_async_copy(
                x_ref.at[pl.ds(off, R)],
                out_ref.at[pl.ds(off, R)],
                copy_sems.at[sem_i],
            )

        for i in range(MAXC):
            @pl.when(i < n_full)
            def _():
                swap_desc(S + i * R, D + i * R, i).start()

        @pl.when(rem > 0)
        def _():
            swap_desc(S + L - R, D + L - R, TAIL).start()

        for i in range(MAXC):
            @pl.when(i < k_full)
            def _():
                keep_desc(K + i * R, i).start()

        @pl.when(k_rem > 0)
        def _():
            keep_desc(K + keep_len - R, TAIL).start()

        for i in range(MAXC):
            @pl.when(i < k_full)
            def _():
                keep_desc(K + i * R, i).wait()

        @pl.when(k_rem > 0)
        def _():
            keep_desc(K + keep_len - R, TAIL).wait()

        for i in range(MAXC):
            @pl.when(i < n_full)
            def _():
                swap_desc(S + i * R, S + i * R, i).wait()

        @pl.when(rem > 0)
        def _():
            swap_desc(S + L - R, S + L - R, TAIL).wait()

    out = pl.pallas_call(
        body,
        out_shape=jax.ShapeDtypeStruct((M, SL, LN), jnp.bfloat16),
        in_specs=[
            pl.BlockSpec(memory_space=pltpu.SMEM),
            pl.BlockSpec(memory_space=pltpu.MemorySpace.HBM),
        ],
        out_specs=pl.BlockSpec(memory_space=pltpu.MemorySpace.HBM),
        scratch_shapes=[
            pltpu.SemaphoreType.DMA((NSEM,)),
            pltpu.SemaphoreType.DMA((NSEM,)),
            pltpu.SemaphoreType.DMA((NSEM,)),
        ],
        compiler_params=pltpu.CompilerParams(collective_id=0),
    )(scal, xs3)
    return out.reshape(M, NC)


# device time: 82751 ns/iter; 1.1635x vs baseline; 1.1635x over previous
import jax
import jax.numpy as jnp
from jax import lax
from jax.experimental import pallas as pl
from jax.experimental.pallas import tpu as pltpu

M = 4096
NC = 1024
SL = 8
LN = NC // SL
R = 256
MAXC = M // R
TAIL = MAXC
NSEM = MAXC + 1


def _cast_bf16(x):
    def body(x_ref, o_ref):
        o_ref[...] = x_ref[...].astype(jnp.bfloat16)

    return pl.pallas_call(
        body,
        grid=(16,),
        in_specs=[pl.BlockSpec((M // 16, NC), lambda i: (i, 0))],
        out_specs=pl.BlockSpec((M // 16, NC), lambda i: (i, 0)),
        out_shape=jax.ShapeDtypeStruct((M, NC), jnp.bfloat16),
    )(x)


def kernel(x, dest):
    p = lax.axis_index("x")
    order = jnp.argsort(dest, stable=True)
    xs = _cast_bf16(x)[order]

    L = jnp.sum(dest == (1 - p)).astype(jnp.int32)
    S = jnp.where(p == 0, M - L, 0)
    D = jnp.where(p == 0, 0, M - L)
    K = jnp.where(p == 0, 0, L)
    scal = jnp.stack([L, S, D, K]).astype(jnp.int32)

    xs3 = xs.reshape(M, SL, LN)

    def body(s_ref, x_ref, out_ref,
             xs_sems, xr_sems, fs_sems, fr_sems, copy_sems):
        my_x = lax.axis_index("x")
        my_y = lax.axis_index("y")
        peer_x = (1 - my_x, my_y)
        peer_y = (my_x, 1 - my_y)

        L = s_ref[0]
        S = s_ref[1]
        D = s_ref[2]
        K = s_ref[3]
        keep_len = M - L
        n_full = L // R
        rem = L - n_full * R
        NH = (n_full + 1) // 2
        k_full = keep_len // R
        k_rem = keep_len - k_full * R
        y0 = my_y == 0

        def mine(k):
            return jnp.where(y0, k < NH, k >= NH) & (k < n_full)

        barrier_sem = pltpu.get_barrier_semaphore()
        for nbr in (peer_x, peer_y):
            pl.semaphore_signal(
                barrier_sem, inc=1, device_id=nbr,
                device_id_type=pl.DeviceIdType.MESH,
            )
        pl.semaphore_wait(barrier_sem, 2)

        def xdesc(off, sem_i):
            return pltpu.make_async_remote_copy(
                src_ref=x_ref.at[pl.ds(S + off, R)],
                dst_ref=out_ref.at[pl.ds(D + off, R)],
                send_sem=xs_sems.at[sem_i],
                recv_sem=xr_sems.at[sem_i],
                device_id=peer_x,
                device_id_type=pl.DeviceIdType.MESH,
            )

        def fdesc(off, sem_i):
            return pltpu.make_async_remote_copy(
                src_ref=out_ref.at[pl.ds(S + off, R)],
                dst_ref=out_ref.at[pl.ds(S + off, R)],
                send_sem=fs_sems.at[sem_i],
                recv_sem=fr_sems.at[sem_i],
                device_id=peer_y,
                device_id_type=pl.DeviceIdType.MESH,
            )

        def keep_desc(off, sem_i):
            return pltpu.make_async_copy(
                x_ref.at[pl.ds(off, R)],
                out_ref.at[pl.ds(off, R)],
                copy_sems.at[sem_i],
            )

        for i in range(MAXC):
            @pl.when(i < k_full)
            def _():
                keep_desc(K + i * R, i).start()

        @pl.when(k_rem > 0)
        def _():
            keep_desc(K + keep_len - R, TAIL).start()

        for k in range(MAXC):
            @pl.when(mine(k))
            def _():
                xdesc(k * R, k).start()

        @pl.when((~y0) & (rem > 0))
        def _():
            xdesc(L - R, TAIL).start()

        for k in range(MAXC):
            @pl.when(mine(k))
            def _():
                d = xdesc(k * R, k)
                d.wait()
                fdesc(k * R, k).start()

        @pl.when((~y0) & (rem > 0))
        def _():
            d = xdesc(L - R, TAIL)
            d.wait()
            fdesc(L - R, TAIL).start()

        for k in range(MAXC):
            @pl.when((~mine(k)) & (k < n_full))
            def _():
                fdesc(k * R, k).wait_recv()

        @pl.when(y0 & (rem > 0))
        def _():
            fdesc(L - R, TAIL).wait_recv()

        for k in range(MAXC):
            @pl.when(mine(k))
            def _():
                fdesc(k * R, k).wait_send()

        @pl.when((~y0) & (rem > 0))
        def _():
            fdesc(L - R, TAIL).wait_send()

        for i in range(MAXC):
            @pl.when(i < k_full)
            def _():
                keep_desc(K + i * R, i).wait()

        @pl.when(k_rem > 0)
        def _():
            keep_desc(K + keep_len - R, TAIL).wait()

    out = pl.pallas_call(
        body,
        out_shape=jax.ShapeDtypeStruct((M, SL, LN), jnp.bfloat16),
        in_specs=[
            pl.BlockSpec(memory_space=pltpu.SMEM),
            pl.BlockSpec(memory_space=pltpu.MemorySpace.HBM),
        ],
        out_specs=pl.BlockSpec(memory_space=pltpu.MemorySpace.HBM),
        scratch_shapes=[
            pltpu.SemaphoreType.DMA((NSEM,)),
            pltpu.SemaphoreType.DMA((NSEM,)),
            pltpu.SemaphoreType.DMA((NSEM,)),
            pltpu.SemaphoreType.DMA((NSEM,)),
            pltpu.SemaphoreType.DMA((NSEM,)),
        ],
        compiler_params=pltpu.CompilerParams(collective_id=0),
    )(scal, xs3)
    return out.reshape(M, NC)


# device time: 78984 ns/iter; 1.2190x vs baseline; 1.0477x over previous
import jax
import jax.numpy as jnp
from jax import lax
from jax.experimental import pallas as pl
from jax.experimental.pallas import tpu as pltpu

M = 4096
NC = 1024
SL = 8
LN = NC // SL
R = 256
MAXC = M // R
TAIL = MAXC
NSEM = MAXC + 1


def _cast_bf16(x):
    def body(x_ref, o_ref):
        o_ref[...] = x_ref[...].astype(jnp.bfloat16)

    return pl.pallas_call(
        body,
        grid=(8,),
        in_specs=[pl.BlockSpec((M // 8, NC), lambda i: (i, 0))],
        out_specs=pl.BlockSpec((M // 8, NC), lambda i: (i, 0)),
        out_shape=jax.ShapeDtypeStruct((M, NC), jnp.bfloat16),
    )(x)


def kernel(x, dest):
    p = lax.axis_index("x")
    order = jnp.argsort(dest, stable=True)
    xs = _cast_bf16(x)[order]

    L = jnp.sum(dest == (1 - p)).astype(jnp.int32)
    S = jnp.where(p == 0, M - L, 0)
    D = jnp.where(p == 0, 0, M - L)
    K = jnp.where(p == 0, 0, L)
    scal = jnp.stack([L, S, D, K]).astype(jnp.int32)

    xs3 = xs.reshape(M, SL, LN)

    def body(s_ref, x_ref, out_ref,
             xs_sems, xr_sems, fs_sems, fr_sems, copy_sems):
        my_x = lax.axis_index("x")
        my_y = lax.axis_index("y")
        peer_x = (1 - my_x, my_y)
        peer_y = (my_x, 1 - my_y)

        L = s_ref[0]
        S = s_ref[1]
        D = s_ref[2]
        K = s_ref[3]
        keep_len = M - L
        n_full = L // R
        rem = L - n_full * R
        NH = (n_full + 1) // 2
        k_full = keep_len // R
        k_rem = keep_len - k_full * R
        y0 = my_y == 0

        def mine(k):
            return jnp.where(y0, k < NH, k >= NH) & (k < n_full)

        barrier_sem = pltpu.get_barrier_semaphore()
        for nbr in (peer_x, peer_y):
            pl.semaphore_signal(
                barrier_sem, inc=1, device_id=nbr,
                device_id_type=pl.DeviceIdType.MESH,
            )
        pl.semaphore_wait(barrier_sem, 2)

        def xdesc(off, sem_i):
            return pltpu.make_async_remote_copy(
                src_ref=x_ref.at[pl.ds(S + off, R)],
                dst_ref=out_ref.at[pl.ds(D + off, R)],
                send_sem=xs_sems.at[sem_i],
                recv_sem=xr_sems.at[sem_i],
                device_id=peer_x,
                device_id_type=pl.DeviceIdType.MESH,
            )

        def fdesc(off, sem_i):
            return pltpu.make_async_remote_copy(
                src_ref=out_ref.at[pl.ds(S + off, R)],
                dst_ref=out_ref.at[pl.ds(S + off, R)],
                send_sem=fs_sems.at[sem_i],
                recv_sem=fr_sems.at[sem_i],
                device_id=peer_y,
                device_id_type=pl.DeviceIdType.MESH,
            )

        def keep_desc(off, sem_i):
            return pltpu.make_async_copy(
                x_ref.at[pl.ds(off, R)],
                out_ref.at[pl.ds(off, R)],
                copy_sems.at[sem_i],
            )

        for i in range(MAXC):
            @pl.when(i < k_full)
            def _():
                keep_desc(K + i * R, i).start()

        @pl.when(k_rem > 0)
        def _():
            keep_desc(K + keep_len - R, TAIL).start()

        for k in range(MAXC):
            @pl.when(mine(k))
            def _():
                xdesc(k * R, k).start()

        @pl.when((~y0) & (rem > 0))
        def _():
            xdesc(L - R, TAIL).start()

        for k in range(MAXC):
            @pl.when(mine(k))
            def _():
                d = xdesc(k * R, k)
                d.wait()
                fdesc(k * R, k).start()

        @pl.when((~y0) & (rem > 0))
        def _():
            d = xdesc(L - R, TAIL)
            d.wait()
            fdesc(L - R, TAIL).start()

        for k in range(MAXC):
            @pl.when((~mine(k)) & (k < n_full))
            def _():
                fdesc(k * R, k).wait_recv()

        @pl.when(y0 & (rem > 0))
        def _():
            fdesc(L - R, TAIL).wait_recv()

        for k in range(MAXC):
            @pl.when(mine(k))
            def _():
                fdesc(k * R, k).wait_send()

        @pl.when((~y0) & (rem > 0))
        def _():
            fdesc(L - R, TAIL).wait_send()

        for i in range(MAXC):
            @pl.when(i < k_full)
            def _():
                keep_desc(K + i * R, i).wait()

        @pl.when(k_rem > 0)
        def _():
            keep_desc(K + keep_len - R, TAIL).wait()

    out = pl.pallas_call(
        body,
        out_shape=jax.ShapeDtypeStruct((M, SL, LN), jnp.bfloat16),
        in_specs=[
            pl.BlockSpec(memory_space=pltpu.SMEM),
            pl.BlockSpec(memory_space=pltpu.MemorySpace.HBM),
        ],
        out_specs=pl.BlockSpec(memory_space=pltpu.MemorySpace.HBM),
        scratch_shapes=[
            pltpu.SemaphoreType.DMA((NSEM,)),
            pltpu.SemaphoreType.DMA((NSEM,)),
            pltpu.SemaphoreType.DMA((NSEM,)),
            pltpu.SemaphoreType.DMA((NSEM,)),
            pltpu.SemaphoreType.DMA((NSEM,)),
        ],
        compiler_params=pltpu.CompilerParams(collective_id=0),
    )(scal, xs3)
    return out.reshape(M, NC)


# device time: 77451 ns/iter; 1.2431x vs baseline; 1.0198x over previous
import jax
import jax.numpy as jnp
from jax import lax
from jax.experimental import pallas as pl
from jax.experimental.pallas import tpu as pltpu

M = 4096
NC = 1024
SL = 8
LN = NC // SL
R = 256
MAXC = M // R
TAIL = MAXC
NSEM = MAXC + 1


def _cast_bf16(x):
    def body(x_ref, o_ref):
        o_ref[...] = x_ref[...].astype(jnp.bfloat16)

    return pl.pallas_call(
        body,
        grid=(4,),
        in_specs=[pl.BlockSpec((M // 4, NC), lambda i: (i, 0))],
        out_specs=pl.BlockSpec((M // 4, NC), lambda i: (i, 0)),
        out_shape=jax.ShapeDtypeStruct((M, NC), jnp.bfloat16),
    )(x)


def kernel(x, dest):
    p = lax.axis_index("x")
    order = jnp.argsort(dest, stable=True)
    xs = _cast_bf16(x)[order]

    L = jnp.sum(dest == (1 - p)).astype(jnp.int32)
    S = jnp.where(p == 0, M - L, 0)
    D = jnp.where(p == 0, 0, M - L)
    K = jnp.where(p == 0, 0, L)
    scal = jnp.stack([L, S, D, K]).astype(jnp.int32)

    xs3 = xs.reshape(M, SL, LN)

    def body(s_ref, x_ref, out_ref,
             xs_sems, xr_sems, fs_sems, fr_sems, copy_sems):
        my_x = lax.axis_index("x")
        my_y = lax.axis_index("y")
        peer_x = (1 - my_x, my_y)
        peer_y = (my_x, 1 - my_y)

        L = s_ref[0]
        S = s_ref[1]
        D = s_ref[2]
        K = s_ref[3]
        keep_len = M - L
        n_full = L // R
        rem = L - n_full * R
        NH = (n_full + 1) // 2
        k_full = keep_len // R
        k_rem = keep_len - k_full * R
        y0 = my_y == 0

        def mine(k):
            return jnp.where(y0, k < NH, k >= NH) & (k < n_full)

        barrier_sem = pltpu.get_barrier_semaphore()
        for nbr in (peer_x, peer_y):
            pl.semaphore_signal(
                barrier_sem, inc=1, device_id=nbr,
                device_id_type=pl.DeviceIdType.MESH,
            )
        pl.semaphore_wait(barrier_sem, 2)

        def xdesc(off, sem_i):
            return pltpu.make_async_remote_copy(
                src_ref=x_ref.at[pl.ds(S + off, R)],
                dst_ref=out_ref.at[pl.ds(D + off, R)],
                send_sem=xs_sems.at[sem_i],
                recv_sem=xr_sems.at[sem_i],
                device_id=peer_x,
                device_id_type=pl.DeviceIdType.MESH,
            )

        def fdesc(off, sem_i):
            return pltpu.make_async_remote_copy(
                src_ref=out_ref.at[pl.ds(S + off, R)],
                dst_ref=out_ref.at[pl.ds(S + off, R)],
                send_sem=fs_sems.at[sem_i],
                recv_sem=fr_sems.at[sem_i],
                device_id=peer_y,
                device_id_type=pl.DeviceIdType.MESH,
            )

        def keep_desc(off, sem_i):
            return pltpu.make_async_copy(
                x_ref.at[pl.ds(off, R)],
                out_ref.at[pl.ds(off, R)],
                copy_sems.at[sem_i],
            )

        for i in range(MAXC):
            @pl.when(i < k_full)
            def _():
                keep_desc(K + i * R, i).start()

        @pl.when(k_rem > 0)
        def _():
            keep_desc(K + keep_len - R, TAIL).start()

        for k in range(MAXC):
            @pl.when(mine(k))
            def _():
                xdesc(k * R, k).start()

        @pl.when((~y0) & (rem > 0))
        def _():
            xdesc(L - R, TAIL).start()

        for k in range(MAXC):
            @pl.when(mine(k))
            def _():
                d = xdesc(k * R, k)
                d.wait()
                fdesc(k * R, k).start()

        @pl.when((~y0) & (rem > 0))
        def _():
            d = xdesc(L - R, TAIL)
            d.wait()
            fdesc(L - R, TAIL).start()

        for k in range(MAXC):
            @pl.when((~mine(k)) & (k < n_full))
            def _():
                fdesc(k * R, k).wait_recv()

        @pl.when(y0 & (rem > 0))
        def _():
            fdesc(L - R, TAIL).wait_recv()

        for k in range(MAXC):
            @pl.when(mine(k))
            def _():
                fdesc(k * R, k).wait_send()

        @pl.when((~y0) & (rem > 0))
        def _():
            fdesc(L - R, TAIL).wait_send()

        for i in range(MAXC):
            @pl.when(i < k_full)
            def _():
                keep_desc(K + i * R, i).wait()

        @pl.when(k_rem > 0)
        def _():
            keep_desc(K + keep_len - R, TAIL).wait()

    out = pl.pallas_call(
        body,
        out_shape=jax.ShapeDtypeStruct((M, SL, LN), jnp.bfloat16),
        in_specs=[
            pl.BlockSpec(memory_space=pltpu.SMEM),
            pl.BlockSpec(memory_space=pltpu.MemorySpace.HBM),
        ],
        out_specs=pl.BlockSpec(memory_space=pltpu.MemorySpace.HBM),
        scratch_shapes=[
            pltpu.SemaphoreType.DMA((NSEM,)),
            pltpu.SemaphoreType.DMA((NSEM,)),
            pltpu.SemaphoreType.DMA((NSEM,)),
            pltpu.SemaphoreType.DMA((NSEM,)),
            pltpu.SemaphoreType.DMA((NSEM,)),
        ],
        compiler_params=pltpu.CompilerParams(collective_id=0),
    )(scal, xs3)
    return out.reshape(M, NC)
